# baseline (speedup 1.0000x reference)
import functools

import jax
import jax.numpy as jnp
from jax import lax
from jax.experimental import pallas as pl
from jax.experimental.pallas import tpu as pltpu
from jax.experimental.pallas import tpu_sc as plsc


@functools.lru_cache(maxsize=None)
def _make_gather(S: int, B: int, V: int, D: int):
    info = plsc.get_sparse_core_info()
    NC, NS = info.num_cores, info.num_subcores
    NW = NC * NS
    N = S * B
    assert N % NW == 0
    rows_per_w = N // NW
    CH = 32
    NB = 3
    nchunk = rows_per_w // CH
    assert rows_per_w % CH == 0

    mesh = plsc.VectorSubcoreMesh(core_axis_name="c", subcore_axis_name="s")

    @functools.partial(
        pl.kernel,
        mesh=mesh,
        out_type=jax.ShapeDtypeStruct((S, B, D), jnp.float32),
        scratch_types=[
            pltpu.VMEM((rows_per_w,), jnp.int32),
        ] + [pltpu.VMEM((CH, D), jnp.float32) for _ in range(NB)]
          + [pltpu.SemaphoreType.DMA for _ in range(2 * NB)],
    )
    def gather_k(idx_hbm, table_hbm, out_hbm, idx_v, *bufs_and_sems):
        bufs = bufs_and_sems[:NB]
        gsems = bufs_and_sems[NB:2 * NB]
        osems = bufs_and_sems[2 * NB:]
        out_flat = out_hbm.reshape(N, D)
        wid = lax.axis_index("s") * NC + lax.axis_index("c")
        base = wid * rows_per_w
        pltpu.sync_copy(idx_hbm.at[pl.ds(base, rows_per_w)], idx_v)

        def start_gather(c):
            b = c % NB
            return pltpu.async_copy(
                table_hbm.at[idx_v.at[pl.ds(c * CH, CH)]], bufs[b], gsems[b])

        def start_outcopy(c):
            b = c % NB
            return pltpu.async_copy(
                bufs[b], out_flat.at[pl.ds(base + c * CH, CH)], osems[b])

        ghandles = [None] * nchunk
        ohandles = [None] * nchunk
        owaited = [False] * nchunk
        for c in range(min(NB, nchunk)):
            ghandles[c] = start_gather(c)
        for c in range(nchunk):
            ghandles[c].wait()
            ohandles[c] = start_outcopy(c)
            nxt = c + NB - 1
            if c >= 1 and nxt < nchunk:
                ohandles[c - 1].wait()
                owaited[c - 1] = True
                ghandles[nxt] = start_gather(nxt)
        for c in range(nchunk):
            if not owaited[c]:
                ohandles[c].wait()

    return gather_k


def _copy_hbm(src_ref, dst_ref, sem):
    pltpu.make_async_copy(src_ref, dst_ref, sem).start()
    pltpu.make_async_copy(src_ref, dst_ref, sem).wait()


@functools.lru_cache(maxsize=None)
def _make_passthrough(B: int, S: int, D: int):
    def run(x):
        return pl.pallas_call(
            _copy_hbm,
            in_specs=[pl.BlockSpec(memory_space=pl.ANY)],
            out_specs=pl.BlockSpec(memory_space=pl.ANY),
            out_shape=jax.ShapeDtypeStruct((B, S, D), jnp.float32),
            scratch_shapes=[pltpu.SemaphoreType.DMA],
        )(x)

    return run


def kernel(enc_hidden_states, dec_tokens, enc_attn_mask, dec_attn_mask,
           enc_dec_attn_mask, dec_labels, table):
    B, S = dec_tokens.shape
    V, D = table.shape
    idx = jnp.transpose(dec_tokens, (1, 0)).reshape(-1).astype(jnp.int32)
    hidden_states = _make_gather(S, B, V, D)(idx, table)
    enc_out = _make_passthrough(B, S, D)(enc_hidden_states)
    return (enc_out, hidden_states)


# speedup vs baseline: 16.3200x; 16.3200x over previous
import functools

import jax
import jax.numpy as jnp
from jax import lax
from jax.experimental import pallas as pl
from jax.experimental.pallas import tpu as pltpu
from jax.experimental.pallas import tpu_sc as plsc


@functools.lru_cache(maxsize=None)
def _make_gather(S: int, B: int, V: int, D: int):
    info = plsc.get_sparse_core_info()
    NC, NS = info.num_cores, info.num_subcores
    NW = NC * NS
    N = S * B
    assert N % NW == 0
    rows_per_w = N // NW
    CH = 32
    NB = 3
    nchunk = rows_per_w // CH
    assert rows_per_w % CH == 0

    mesh = plsc.VectorSubcoreMesh(core_axis_name="c", subcore_axis_name="s")

    @functools.partial(
        pl.kernel,
        mesh=mesh,
        out_type=jax.ShapeDtypeStruct((S, B, D), jnp.float32),
        scratch_types=[
            pltpu.VMEM((rows_per_w,), jnp.int32),
        ] + [pltpu.VMEM((CH, D), jnp.float32) for _ in range(NB)]
          + [pltpu.SemaphoreType.DMA for _ in range(2 * NB)],
    )
    def gather_k(idx_hbm, table_hbm, out_hbm, idx_v, *bufs_and_sems):
        bufs = bufs_and_sems[:NB]
        gsems = bufs_and_sems[NB:2 * NB]
        osems = bufs_and_sems[2 * NB:]
        out_flat = out_hbm.reshape(N, D)
        wid = lax.axis_index("s") * NC + lax.axis_index("c")
        base = wid * rows_per_w
        pltpu.sync_copy(idx_hbm.at[pl.ds(base, rows_per_w)], idx_v)

        def start_gather(c):
            b = c % NB
            return pltpu.async_copy(
                table_hbm.at[idx_v.at[pl.ds(c * CH, CH)]], bufs[b], gsems[b])

        def start_outcopy(c):
            b = c % NB
            return pltpu.async_copy(
                bufs[b], out_flat.at[pl.ds(base + c * CH, CH)], osems[b])

        ghandles = [None] * nchunk
        ohandles = [None] * nchunk
        owaited = [False] * nchunk
        for c in range(min(NB, nchunk)):
            ghandles[c] = start_gather(c)
        for c in range(nchunk):
            ghandles[c].wait()
            ohandles[c] = start_outcopy(c)
            nxt = c + NB - 1
            if c >= 1 and nxt < nchunk:
                ohandles[c - 1].wait()
                owaited[c - 1] = True
                ghandles[nxt] = start_gather(nxt)
        for c in range(nchunk):
            if not owaited[c]:
                ohandles[c].wait()

    return gather_k


@functools.lru_cache(maxsize=None)
def _make_passthrough(B: int, S: int, D: int):
    N = B * S
    BLK = 512
    TNB = 4
    nchunk = N // BLK
    assert N % BLK == 0

    def body(src_ref, dst_ref, *bufs_and_sems):
        bufs = bufs_and_sems[:TNB]
        isems = bufs_and_sems[TNB:2 * TNB]
        osems = bufs_and_sems[2 * TNB:]

        def start_in(c):
            b = c % TNB
            return pltpu.make_async_copy(
                src_ref.at[pl.ds(c * BLK, BLK)], bufs[b], isems[b])

        def start_out(c):
            b = c % TNB
            return pltpu.make_async_copy(
                bufs[b], dst_ref.at[pl.ds(c * BLK, BLK)], osems[b])

        ih = [None] * nchunk
        oh = [None] * nchunk
        ow = [False] * nchunk
        for c in range(min(TNB, nchunk)):
            ih[c] = start_in(c)
            ih[c].start()
        for c in range(nchunk):
            ih[c].wait()
            oh[c] = start_out(c)
            oh[c].start()
            nxt = c + TNB - 1
            if c >= 1 and nxt < nchunk:
                oh[c - 1].wait()
                ow[c - 1] = True
                ih[nxt] = start_in(nxt)
                ih[nxt].start()
        for c in range(nchunk):
            if not ow[c]:
                oh[c].wait()

    def run(x):
        flat = x.reshape(N, D)
        out = pl.pallas_call(
            body,
            in_specs=[pl.BlockSpec(memory_space=pl.ANY)],
            out_specs=pl.BlockSpec(memory_space=pl.ANY),
            out_shape=jax.ShapeDtypeStruct((N, D), jnp.float32),
            scratch_shapes=[pltpu.VMEM((BLK, D), jnp.float32)
                            for _ in range(TNB)]
                           + [pltpu.SemaphoreType.DMA for _ in range(2 * TNB)],
        )(flat)
        return out.reshape(B, S, D)

    return run


def kernel(enc_hidden_states, dec_tokens, enc_attn_mask, dec_attn_mask,
           enc_dec_attn_mask, dec_labels, table):
    B, S = dec_tokens.shape
    V, D = table.shape
    idx = jnp.transpose(dec_tokens, (1, 0)).reshape(-1).astype(jnp.int32)
    hidden_states = _make_gather(S, B, V, D)(idx, table)
    enc_out = _make_passthrough(B, S, D)(enc_hidden_states)
    return (enc_out, hidden_states)


# speedup vs baseline: 16.5622x; 1.0148x over previous
import functools

import jax
import jax.numpy as jnp
from jax import lax
from jax.experimental import pallas as pl
from jax.experimental.pallas import tpu as pltpu
from jax.experimental.pallas import tpu_sc as plsc


@functools.lru_cache(maxsize=None)
def _make_gather(S: int, B: int, V: int, D: int):
    info = plsc.get_sparse_core_info()
    NC, NS = info.num_cores, info.num_subcores
    NW = NC * NS
    N = S * B
    assert N % NW == 0
    rows_per_w = N // NW
    s_per_w = S // NW
    HCH = 32
    NB = 3
    nchunk = rows_per_w // HCH
    assert s_per_w % HCH == 0

    mesh = plsc.VectorSubcoreMesh(core_axis_name="c", subcore_axis_name="s")

    @functools.partial(
        pl.kernel,
        mesh=mesh,
        out_type=jax.ShapeDtypeStruct((S, B, D), jnp.float32),
        scratch_types=[
            pltpu.VMEM((B, s_per_w), jnp.int32),
        ] + [pltpu.VMEM((HCH, D), jnp.float32) for _ in range(NB)]
          + [pltpu.SemaphoreType.DMA for _ in range(2 * NB)],
    )
    def gather_k(tok_hbm, table_hbm, out_hbm, idx_v, *bufs_and_sems):
        bufs = bufs_and_sems[:NB]
        gsems = bufs_and_sems[NB:2 * NB]
        osems = bufs_and_sems[2 * NB:]
        out_v = out_hbm.reshape(NW, s_per_w, B, D)
        wid = lax.axis_index("s") * NC + lax.axis_index("c")
        s_base = wid * s_per_w
        for bt in range(B):
            pltpu.sync_copy(tok_hbm.at[bt, pl.ds(s_base, s_per_w)],
                            idx_v.at[bt])

        def start_gather(c):
            b = c % NB
            bt, h = divmod(c, s_per_w // HCH)
            return pltpu.async_copy(
                table_hbm.at[idx_v.at[bt, pl.ds(h * HCH, HCH)]],
                bufs[b], gsems[b])

        def start_outcopy(c):
            b = c % NB
            bt, h = divmod(c, s_per_w // HCH)
            return pltpu.async_copy(
                bufs[b], out_v.at[wid, pl.ds(h * HCH, HCH), bt], osems[b])

        ghandles = [None] * nchunk
        ohandles = [None] * nchunk
        owaited = [False] * nchunk
        for c in range(min(NB, nchunk)):
            ghandles[c] = start_gather(c)
        for c in range(nchunk):
            ghandles[c].wait()
            ohandles[c] = start_outcopy(c)
            nxt = c + NB - 1
            if c >= 1 and nxt < nchunk:
                ohandles[c - 1].wait()
                owaited[c - 1] = True
                ghandles[nxt] = start_gather(nxt)
        for c in range(nchunk):
            if not owaited[c]:
                ohandles[c].wait()

    return gather_k


@functools.lru_cache(maxsize=None)
def _make_passthrough(B: int, S: int, D: int):
    N = B * S
    BLK = 512
    TNB = 4
    nchunk = N // BLK
    assert N % BLK == 0

    def body(src_ref, dst_ref, *bufs_and_sems):
        bufs = bufs_and_sems[:TNB]
        isems = bufs_and_sems[TNB:2 * TNB]
        osems = bufs_and_sems[2 * TNB:]

        def start_in(c):
            b = c % TNB
            return pltpu.make_async_copy(
                src_ref.at[pl.ds(c * BLK, BLK)], bufs[b], isems[b])

        def start_out(c):
            b = c % TNB
            return pltpu.make_async_copy(
                bufs[b], dst_ref.at[pl.ds(c * BLK, BLK)], osems[b])

        ih = [None] * nchunk
        oh = [None] * nchunk
        ow = [False] * nchunk
        for c in range(min(TNB, nchunk)):
            ih[c] = start_in(c)
            ih[c].start()
        for c in range(nchunk):
            ih[c].wait()
            oh[c] = start_out(c)
            oh[c].start()
            nxt = c + TNB - 1
            if c >= 1 and nxt < nchunk:
                oh[c - 1].wait()
                ow[c - 1] = True
                ih[nxt] = start_in(nxt)
                ih[nxt].start()
        for c in range(nchunk):
            if not ow[c]:
                oh[c].wait()

    def run(x):
        flat = x.reshape(N, D)
        out = pl.pallas_call(
            body,
            in_specs=[pl.BlockSpec(memory_space=pl.ANY)],
            out_specs=pl.BlockSpec(memory_space=pl.ANY),
            out_shape=jax.ShapeDtypeStruct((N, D), jnp.float32),
            scratch_shapes=[pltpu.VMEM((BLK, D), jnp.float32)
                            for _ in range(TNB)]
                           + [pltpu.SemaphoreType.DMA for _ in range(2 * TNB)],
        )(flat)
        return out.reshape(B, S, D)

    return run


def kernel(enc_hidden_states, dec_tokens, enc_attn_mask, dec_attn_mask,
           enc_dec_attn_mask, dec_labels, table):
    B, S = dec_tokens.shape
    V, D = table.shape
    tok = dec_tokens if dec_tokens.dtype == jnp.int32 else (
        dec_tokens.astype(jnp.int32))
    hidden_states = _make_gather(S, B, V, D)(tok, table)
    enc_out = _make_passthrough(B, S, D)(enc_hidden_states)
    return (enc_out, hidden_states)
